# Initial kernel scaffold; baseline (speedup 1.0000x reference)
import jax
import jax.numpy as jnp
from jax.experimental import pallas as pl


def kernel(x_user, x_item, edge_index_rates, edge_index_rated_by, W1_rates_self, W1_rates_neigh, W1_rb_self, W1_rb_neigh, W2_rates_self, W2_rates_neigh, W2_rb_self, W2_rb_neigh, b1_rates, b1_rb, b2_rates, b2_rb):
    raise NotImplementedError("write your pallas kernel here")


# speedup vs baseline: 16.5578x; 16.5578x over previous
import jax
import jax.numpy as jnp
from jax import lax
from jax.experimental import pallas as pl
from jax.experimental.pallas import tpu as pltpu
from jax.experimental.pallas import tpu_sc as plsc

N = 100000
D = 32
DH = 16
E = 1600000
CHUNK = 128
SUBS = 8
NTILES = 16
NSUP = 98
KITER = NSUP // 2
NSUP_TOT = NTILES * NSUP
EPAD = NSUP_TOT * SUBS * CHUNK
ACC_ROWS = 100096
RPT = ACC_ROWS // NTILES
ZROWS = 782

_mesh = plsc.VectorSubcoreMesh(core_axis_name="c", subcore_axis_name="s")
_sc_params = pltpu.CompilerParams(use_tc_tiling_on_sc=False)


def _zero_acc(zbuf, acc, t):
    def fill_zero(i, carry):
        zbuf[i, :] = jnp.zeros((DH,), jnp.float32)
        return carry

    lax.fori_loop(0, ZROWS, fill_zero, 0)

    def zero_copy(k, carry):
        pltpu.sync_copy(zbuf, acc.at[pl.ds(t * RPT + k * ZROWS, ZROWS)])
        return carry

    lax.fori_loop(0, RPT // ZROWS, zero_copy, 0)


def _deg_body(eidx_r_hbm, eidx_b_hbm, deg_r_hbm, deg_b_hbm,
              eA, eB, ones_v, zbuf, acc, isemA, isemB, ssemA, ssemB):
    c = lax.axis_index("c")
    t = lax.axis_index("s")

    def fill_ones(i, carry):
        ones_v[i, :] = jnp.ones((DH,), jnp.float32)
        return carry

    lax.fori_loop(0, CHUNK, fill_ones, 0)
    _zero_acc(zbuf, acc, t)
    plsc.subcore_barrier()

    def run(eidx_hbm):
        base = t * NSUP

        def scatters(e, sem):
            for j in range(SUBS):
                pltpu.async_copy(ones_v, acc.at[e.at[SUBS + j]], sem, add=True)

        def wait_scatters(e, sem):
            for j in range(SUBS):
                pltpu.make_async_copy(ones_v, acc.at[e.at[SUBS + j]], sem).wait()

        pltpu.sync_copy(eidx_hbm.at[base], eA)

        def body(k, carry):
            gB = 2 * k + 1

            @pl.when(k > 0)
            def _():
                wait_scatters(eB, ssemB)

            idx_b = pltpu.async_copy(eidx_hbm.at[base + gB], eB, isemB)
            scatters(eA, ssemA)
            idx_b.wait()
            wait_scatters(eA, ssemA)

            @pl.when(k < KITER - 1)
            def _():
                pltpu.async_copy(eidx_hbm.at[base + gB + 1], eA, isemA)

            scatters(eB, ssemB)

            @pl.when(k < KITER - 1)
            def _():
                pltpu.make_async_copy(eidx_hbm.at[base + gB + 1], eA,
                                      isemA).wait()

            return carry

        lax.fori_loop(0, KITER, body, 0)
        wait_scatters(eB, ssemB)

    @pl.when(c == 0)
    def _():
        run(eidx_r_hbm)

    @pl.when(c == 1)
    def _():
        run(eidx_b_hbm)

    plsc.subcore_barrier()

    @pl.when(c == 0)
    def _():
        pltpu.sync_copy(acc.at[pl.ds(t * RPT, RPT)],
                        deg_r_hbm.at[pl.ds(t * RPT, RPT)])

    @pl.when(c == 1)
    def _():
        pltpu.sync_copy(acc.at[pl.ds(t * RPT, RPT)],
                        deg_b_hbm.at[pl.ds(t * RPT, RPT)])


_deg_call = pl.kernel(
    _deg_body,
    out_type=(jax.ShapeDtypeStruct((ACC_ROWS, DH), jnp.float32),
              jax.ShapeDtypeStruct((ACC_ROWS, DH), jnp.float32)),
    mesh=_mesh,
    scratch_types=[
        pltpu.VMEM((2 * SUBS, CHUNK), jnp.int32),
        pltpu.VMEM((2 * SUBS, CHUNK), jnp.int32),
        pltpu.VMEM((CHUNK, DH), jnp.float32),
        pltpu.VMEM((ZROWS, DH), jnp.float32),
        pltpu.VMEM_SHARED((ACC_ROWS, DH), jnp.float32),
        pltpu.SemaphoreType.DMA,
        pltpu.SemaphoreType.DMA,
        pltpu.SemaphoreType.DMA,
        pltpu.SemaphoreType.DMA,
    ],
    compiler_params=_sc_params,
)


def _agg_body(tlo_hbm, thi_hbm, eidx_hbm, out_lo_hbm, out_hi_hbm,
              eA, eB, rows0, rows1, rows2, rows3, zbuf, acc,
              isemA, isemB, gs0, gs1, gs2, gs3, ss0, ss1, ss2, ss3):
    c = lax.axis_index("c")
    t = lax.axis_index("s")
    rows = (rows0, rows1, rows2, rows3)
    gsem = (gs0, gs1, gs2, gs3)
    ssem = (ss0, ss1, ss2, ss3)
    _zero_acc(zbuf, acc, t)
    plsc.subcore_barrier()

    def run(table):
        base = t * NSUP

        def srow(e, m):
            return e.at[SUBS + m]

        def g_issue(e, m, p):
            pltpu.async_copy(table.at[e.at[m]], rows[p], gsem[p])

        def g_wait(e, m, p):
            pltpu.make_async_copy(table.at[e.at[m]], rows[p], gsem[p]).wait()

        def s_issue(e, m, p):
            pltpu.async_copy(rows[p], acc.at[srow(e, m)], ssem[p], add=True)

        def s_wait(e, m, p):
            pltpu.make_async_copy(rows[p], acc.at[srow(e, m)], ssem[p]).wait()

        pltpu.sync_copy(eidx_hbm.at[base], eA)
        g_issue(eA, 0, 0)
        g_issue(eA, 1, 1)

        def body(k, carry):
            for j in range(16):
                p = j % 4
                q = (j + 2) % 4
                if j < 2:
                    @pl.when(k > 0)
                    def _(j=j, q=q):
                        s_wait(eB, j + 14 - 8, q)
                else:
                    jm = j - 2
                    s_wait(eA if jm < 8 else eB, jm % 8, q)
                if j == 2:
                    pltpu.async_copy(eidx_hbm.at[base + 2 * k + 1], eB, isemB)
                if j == 6:
                    pltpu.make_async_copy(eidx_hbm.at[base + 2 * k + 1], eB,
                                          isemB).wait()
                jp = j + 2
                if jp < 8:
                    g_issue(eA, jp, q)
                elif jp < 16:
                    g_issue(eB, jp - 8, q)
                else:
                    if j == 14:
                        @pl.when(k < KITER - 1)
                        def _():
                            pltpu.make_async_copy(
                                eidx_hbm.at[base + 2 * k + 2], eA,
                                isemA).wait()
                            g_issue(eA, 0, 0)
                    else:
                        @pl.when(k < KITER - 1)
                        def _():
                            g_issue(eA, 1, 1)
                if j < 8:
                    g_wait(eA, j, p)
                    s_issue(eA, j, p)
                else:
                    g_wait(eB, j - 8, p)
                    s_issue(eB, j - 8, p)
                if j == 9:
                    @pl.when(k < KITER - 1)
                    def _():
                        pltpu.async_copy(eidx_hbm.at[base + 2 * k + 2], eA,
                                         isemA)
            return carry

        lax.fori_loop(0, KITER, body, 0)
        s_wait(eB, 6, 2)
        s_wait(eB, 7, 3)

    @pl.when(c == 0)
    def _():
        run(tlo_hbm)

    @pl.when(c == 1)
    def _():
        run(thi_hbm)

    plsc.subcore_barrier()

    @pl.when(c == 0)
    def _():
        pltpu.sync_copy(acc.at[pl.ds(t * RPT, RPT)],
                        out_lo_hbm.at[pl.ds(t * RPT, RPT)])

    @pl.when(c == 1)
    def _():
        pltpu.sync_copy(acc.at[pl.ds(t * RPT, RPT)],
                        out_hi_hbm.at[pl.ds(t * RPT, RPT)])


_agg_call = pl.kernel(
    _agg_body,
    out_type=(jax.ShapeDtypeStruct((ACC_ROWS, DH), jnp.float32),
              jax.ShapeDtypeStruct((ACC_ROWS, DH), jnp.float32)),
    mesh=_mesh,
    scratch_types=[
        pltpu.VMEM((2 * SUBS, CHUNK), jnp.int32),
        pltpu.VMEM((2 * SUBS, CHUNK), jnp.int32),
        pltpu.VMEM((CHUNK, DH), jnp.float32),
        pltpu.VMEM((CHUNK, DH), jnp.float32),
        pltpu.VMEM((CHUNK, DH), jnp.float32),
        pltpu.VMEM((CHUNK, DH), jnp.float32),
        pltpu.VMEM((ZROWS, DH), jnp.float32),
        pltpu.VMEM_SHARED((ACC_ROWS, DH), jnp.float32),
    ] + [pltpu.SemaphoreType.DMA] * 10,
    compiler_params=_sc_params,
)

ROWS_TC = 2000
GRID_TC = N // ROWS_TC


def _tc1_body(x_ref, lo_ref, hi_ref, deg_ref, ws_ref, wl_ref, wh_ref, b_ref,
              out_lo_ref, out_hi_ref):
    agg = (jnp.dot(lo_ref[...], wl_ref[...], preferred_element_type=jnp.float32)
           + jnp.dot(hi_ref[...], wh_ref[...], preferred_element_type=jnp.float32))
    inv = 1.0 / jnp.maximum(deg_ref[...][:, :1], 1.0)
    h = (jnp.dot(x_ref[...], ws_ref[...], preferred_element_type=jnp.float32)
         + inv * agg + b_ref[...])
    h = jnp.maximum(h, 0.0)
    out_lo_ref[...] = h[:, :DH]
    out_hi_ref[...] = h[:, DH:]


def _tc2_body(xlo_ref, xhi_ref, lo_ref, hi_ref, deg_ref,
              wslo_ref, wshi_ref, wl_ref, wh_ref, b_ref, out_ref):
    agg = (jnp.dot(lo_ref[...], wl_ref[...], preferred_element_type=jnp.float32)
           + jnp.dot(hi_ref[...], wh_ref[...], preferred_element_type=jnp.float32))
    inv = 1.0 / jnp.maximum(deg_ref[...][:, :1], 1.0)
    h = (jnp.dot(xlo_ref[...], wslo_ref[...], preferred_element_type=jnp.float32)
         + jnp.dot(xhi_ref[...], wshi_ref[...], preferred_element_type=jnp.float32)
         + inv * agg + b_ref[...])
    out_ref[...] = h


def _row_spec(cols):
    return pl.BlockSpec((ROWS_TC, cols), lambda i: (i, 0))


def _full_spec(r, c):
    return pl.BlockSpec((r, c), lambda i: (0, 0))


_tc1_call = pl.pallas_call(
    _tc1_body,
    grid=(GRID_TC,),
    in_specs=[_row_spec(D), _row_spec(DH), _row_spec(DH), _row_spec(DH),
              _full_spec(D, D), _full_spec(DH, D), _full_spec(DH, D),
              _full_spec(1, D)],
    out_specs=(_row_spec(DH), _row_spec(DH)),
    out_shape=(jax.ShapeDtypeStruct((N, DH), jnp.float32),
               jax.ShapeDtypeStruct((N, DH), jnp.float32)),
)

_tc2_call = pl.pallas_call(
    _tc2_body,
    grid=(GRID_TC,),
    in_specs=[_row_spec(DH), _row_spec(DH), _row_spec(DH), _row_spec(DH),
              _row_spec(DH), _full_spec(DH, D), _full_spec(DH, D),
              _full_spec(DH, D), _full_spec(DH, D), _full_spec(1, D)],
    out_specs=_row_spec(D),
    out_shape=jax.ShapeDtypeStruct((N, D), jnp.float32),
)


def _edge_blocks(ei):
    src = jnp.concatenate([ei[0].astype(jnp.int32),
                           jnp.zeros((EPAD - E,), jnp.int32)])
    dst = jnp.concatenate([ei[1].astype(jnp.int32),
                           jnp.full((EPAD - E,), N, jnp.int32)])
    s3 = src.reshape(NSUP_TOT, SUBS, CHUNK)
    d3 = dst.reshape(NSUP_TOT, SUBS, CHUNK)
    return jnp.concatenate([s3, d3], axis=1)


def kernel(x_user, x_item, edge_index_rates, edge_index_rated_by,
           W1_rates_self, W1_rates_neigh, W1_rb_self, W1_rb_neigh,
           W2_rates_self, W2_rates_neigh, W2_rb_self, W2_rb_neigh,
           b1_rates, b1_rb, b2_rates, b2_rb):
    e_r = _edge_blocks(edge_index_rates)
    e_b = _edge_blocks(edge_index_rated_by)

    xu_lo, xu_hi = x_user[:, :DH], x_user[:, DH:]
    xi_lo, xi_hi = x_item[:, :DH], x_item[:, DH:]

    deg_r, deg_b = _deg_call(e_r, e_b)

    b1r = b1_rates.reshape(1, D)
    b1b = b1_rb.reshape(1, D)
    b2r = b2_rates.reshape(1, D)
    b2b = b2_rb.reshape(1, D)

    a1i_lo, a1i_hi = _agg_call(xu_lo, xu_hi, e_r)
    a1u_lo, a1u_hi = _agg_call(xi_lo, xi_hi, e_b)
    hi_lo, hi_hi = _tc1_call(x_item, a1i_lo, a1i_hi, deg_r,
                             W1_rates_self, W1_rates_neigh[:DH],
                             W1_rates_neigh[DH:], b1r)
    hu_lo, hu_hi = _tc1_call(x_user, a1u_lo, a1u_hi, deg_b,
                             W1_rb_self, W1_rb_neigh[:DH],
                             W1_rb_neigh[DH:], b1b)

    a2i_lo, a2i_hi = _agg_call(hu_lo, hu_hi, e_r)
    a2u_lo, a2u_hi = _agg_call(hi_lo, hi_hi, e_b)
    h_item2 = _tc2_call(hi_lo, hi_hi, a2i_lo, a2i_hi, deg_r,
                        W2_rates_self[:DH], W2_rates_self[DH:],
                        W2_rates_neigh[:DH], W2_rates_neigh[DH:], b2r)
    h_user2 = _tc2_call(hu_lo, hu_hi, a2u_lo, a2u_hi, deg_b,
                        W2_rb_self[:DH], W2_rb_self[DH:],
                        W2_rb_neigh[:DH], W2_rb_neigh[DH:], b2b)
    return (h_user2, h_item2)
